# baseline (speedup 1.0000x reference)
import functools

import jax
import jax.numpy as jnp
from jax import lax
from jax.experimental import pallas as pl
from jax.experimental.pallas import tpu as pltpu
from jax.experimental.pallas import tpu_sc as plsc

N = 10000
E = 320000
HID = 128
HH = HID // 2
ATTR = 16
NB = 8
NCONV = 3
GAMMA = 10.0

NP = 10240
NC, NS = 2, 16
NW = NC * NS
CH = 80
K = 2
KA = 5
ECT = E // NW
NCH = ECT // CH
ECS = E // NS
NCHS = ECS // CH
XCT = NP // NW
XCH = 4
XCS = XCT // XCH
RPT = NP // NS

C0 = 0.28209479177387814
C1 = 0.4886025119029199


def _silu(v):
    return v / (1.0 + jnp.exp(-v))


def _sc_gather_body(pos_hbm, src3_hbm, dst3_hbm, x3_hbm, emb_hbm,
                    dvec_hbm, xattr_hbm,
                    src_t, dst_t, x_t, psrc, pdst, obuf, xbuf,
                    lsem, wsem):
    c = lax.axis_index("c")
    s = lax.axis_index("s")
    wid = c * NS + s
    ebase = wid * ECT

    pltpu.sync_copy(src3_hbm.at[wid], src_t)
    pltpu.sync_copy(dst3_hbm.at[wid], dst_t)
    pltpu.sync_copy(x3_hbm.at[wid], x_t)

    def group(gi, _):
        i0 = gi * KA
        descs = []
        for b in range(KA):
            descs.append(pltpu.async_copy(
                pos_hbm.at[src_t.at[i0 + b]], psrc.at[b], lsem))
            descs.append(pltpu.async_copy(
                pos_hbm.at[dst_t.at[i0 + b]], pdst.at[b], lsem))
        for d in descs:
            d.wait()
        wdescs = []
        for b in range(KA):
            def row(r4, _):
                for rr in range(4):
                    r = r4 * 4 + rr
                    obuf[b, r] = pdst[b, r] - psrc[b, r]
                return 0

            lax.fori_loop(0, CH // 4, row, 0)
            wdescs.append(pltpu.async_copy(
                obuf.at[b], dvec_hbm.at[pl.ds(ebase + (i0 + b) * CH, CH)],
                wsem))
        for d in wdescs:
            d.wait()
        return 0

    lax.fori_loop(0, NCH // KA, group, 0)

    xbase = wid * XCT

    def xchunk(k, _):
        pltpu.async_copy(emb_hbm.at[x_t.at[k]], xbuf, lsem).wait()
        pltpu.sync_copy(xbuf, xattr_hbm.at[pl.ds(xbase + k * XCS, XCS)])
        return 0

    lax.fori_loop(0, XCH, xchunk, 0)


@functools.cache
def _make_sc_gather():
  return pl.kernel(
    _sc_gather_body,
    out_type=(jax.ShapeDtypeStruct((E, 16), jnp.float32),
              jax.ShapeDtypeStruct((NP, ATTR), jnp.float32)),
    mesh=plsc.VectorSubcoreMesh(core_axis_name="c", subcore_axis_name="s"),
    compiler_params=pltpu.CompilerParams(use_tc_tiling_on_sc=False),
    scratch_types=(
        pltpu.VMEM((NCH, CH), jnp.int32),
        pltpu.VMEM((NCH, CH), jnp.int32),
        pltpu.VMEM((XCH, XCS), jnp.int32),
        pltpu.VMEM((KA, CH, 16), jnp.float32),
        pltpu.VMEM((KA, CH, 16), jnp.float32),
        pltpu.VMEM((KA, CH, 16), jnp.float32),
        pltpu.VMEM((XCS, ATTR), jnp.float32),
        pltpu.SemaphoreType.DMA,
        pltpu.SemaphoreType.DMA,
    ),
  )


def _sc_agg_body(h_hbm, g_hbm, src3_hbm, dst3_hbm,
                 out_hbm,
                 src_t, dst_t, hb0, hb1, gb0, gb1, zbuf, agg_s,
                 lsem, ssem):
    hbl = (hb0, hb1)
    gbl = (gb0, gb1)
    c = lax.axis_index("c")
    s = lax.axis_index("s")
    ebase = s * ECS

    zv = jnp.zeros((16,), jnp.float32)

    def zrow(r, _):
        for j in range(HH // 16):
            zbuf[r, pl.ds(j * 16, 16)] = zv
        return 0

    lax.fori_loop(0, CH, zrow, 0)
    for t in range(RPT // CH):
        pltpu.sync_copy(zbuf, agg_s.at[pl.ds(s * RPT + t * CH, CH)])
    plsc.subcore_barrier()

    pltpu.sync_copy(src3_hbm.at[s], src_t)
    pltpu.sync_copy(dst3_hbm.at[s], dst_t)

    def group(gi, _):
        i0 = gi * K
        descs = []
        for b in range(K):
            descs.append(pltpu.async_copy(
                h_hbm.at[c].at[src_t.at[i0 + b]], hbl[b], lsem))
            descs.append(pltpu.async_copy(
                g_hbm.at[c, pl.ds(ebase + (i0 + b) * CH, CH)],
                gbl[b], lsem))
        for d in descs:
            d.wait()
        for b in range(K):
            def row(r2, _):
                for rr in range(2):
                    r = r2 * 2 + rr
                    for j in range(HH // 32):
                        v = plsc.bitcast(gbl[b][r, pl.ds(j * 32, 32)],
                                         jnp.int32)
                        a0 = plsc.bitcast(v << 16, jnp.float32)
                        a1 = plsc.bitcast(v & jnp.int32(-65536), jnp.float32)
                        sl0 = pl.ds(j * 32, 16)
                        sl1 = pl.ds(j * 32 + 16, 16)
                        hbl[b][r, sl0] = hbl[b][r, sl0] * a0
                        hbl[b][r, sl1] = hbl[b][r, sl1] * a1
                return 0

            lax.fori_loop(0, CH // 2, row, 0)
            pltpu.sync_copy(hbl[b], agg_s.at[dst_t.at[i0 + b]], add=True)
        return 0

    lax.fori_loop(0, NCHS // K, group, 0)
    plsc.subcore_barrier()
    pltpu.sync_copy(agg_s.at[pl.ds(s * RPT, RPT)],
                    out_hbm.at[c, pl.ds(s * RPT, RPT)])


@functools.cache
def _make_sc_agg():
  return pl.kernel(
    _sc_agg_body,
    out_type=jax.ShapeDtypeStruct((NC, NP, HH), jnp.float32),
    mesh=plsc.VectorSubcoreMesh(core_axis_name="c", subcore_axis_name="s"),
    compiler_params=pltpu.CompilerParams(use_tc_tiling_on_sc=False,
                                         needs_layout_passes=False),
    scratch_types=(
        pltpu.VMEM((NCHS, CH), jnp.int32),
        pltpu.VMEM((NCHS, CH), jnp.int32),
        pltpu.VMEM((CH, HH), jnp.float32),
        pltpu.VMEM((CH, HH), jnp.float32),
        pltpu.VMEM((CH, HH), jnp.bfloat16),
        pltpu.VMEM((CH, HH), jnp.bfloat16),
        pltpu.VMEM((CH, HH), jnp.float32),
        pltpu.VMEM_SHARED((NP, HH), jnp.float32),
        pltpu.SemaphoreType.DMA,
        pltpu.SemaphoreType.DMA,
    ),
  )


BE = 2000


def _gate_body(dvec_ref, per_ref, mu_ref, W1_ref, W2_ref, Wsh0_ref, Wshp_ref,
               g_ref):
    dv = dvec_ref[...] + per_ref[...]
    r2 = jnp.sum(dv * dv, axis=1, keepdims=True) + 1e-12
    r = jnp.sqrt(r2)
    up = dv / r
    rbf = jnp.exp(-GAMMA * (r - mu_ref[...]) ** 2)
    q = _silu(jnp.dot(rbf, W1_ref[...], preferred_element_type=jnp.float32))
    rad = jnp.dot(q, W2_ref[...], preferred_element_type=jnp.float32)
    shw = C0 * Wsh0_ref[...] + C1 * jnp.dot(
        up, Wshp_ref[...], preferred_element_type=jnp.float32)
    gz = (rad * shw).astype(jnp.bfloat16)
    g_ref[...] = jnp.stack([gz[:, :HH], gz[:, HH:]])


_gates1 = pl.pallas_call(
    _gate_body,
    grid=(E // BE,),
    in_specs=[
        pl.BlockSpec((BE, 16), lambda i: (i, 0)),
        pl.BlockSpec((BE, 16), lambda i: (i, 0)),
        pl.BlockSpec((1, NB), lambda i: (0, 0)),
        pl.BlockSpec((NB, HID), lambda i: (0, 0)),
        pl.BlockSpec((HID, HID), lambda i: (0, 0)),
        pl.BlockSpec((1, HID), lambda i: (0, 0)),
        pl.BlockSpec((16, HID), lambda i: (0, 0)),
    ],
    out_specs=pl.BlockSpec((NC, BE, HH), lambda i: (0, i, 0)),
    out_shape=jax.ShapeDtypeStruct((NC, E, HH), jnp.bfloat16),
)


def _h0_body(xattr_ref, w_ref, h_ref):
    h = jnp.dot(xattr_ref[...], w_ref[...], preferred_element_type=jnp.float32)
    h_ref[...] = jnp.stack([h[:, :HH], h[:, HH:]])


_h0 = pl.pallas_call(
    _h0_body,
    out_shape=jax.ShapeDtypeStruct((NC, NP, HH), jnp.float32),
)


BN = 1280


def _update_body(h_ref, agg_ref, xattr_ref, ws_ref, wo_ref, wa_ref, out_ref):
    h = jnp.concatenate([h_ref[0], h_ref[1]], axis=1)
    agg = jnp.concatenate([agg_ref[0], agg_ref[1]], axis=1)
    v = (jnp.dot(h, ws_ref[...], preferred_element_type=jnp.float32)
         + jnp.dot(agg, wo_ref[...], preferred_element_type=jnp.float32)
         + jnp.dot(xattr_ref[...], wa_ref[...],
                   preferred_element_type=jnp.float32))
    hn = _silu(v)
    out_ref[...] = jnp.stack([hn[:, :HH], hn[:, HH:]])


_update = pl.pallas_call(
    _update_body,
    grid=(NP // BN,),
    in_specs=[
        pl.BlockSpec((NC, BN, HH), lambda i: (0, i, 0)),
        pl.BlockSpec((NC, BN, HH), lambda i: (0, i, 0)),
        pl.BlockSpec((BN, ATTR), lambda i: (i, 0)),
        pl.BlockSpec((HID, HID), lambda i: (0, 0)),
        pl.BlockSpec((HID, HID), lambda i: (0, 0)),
        pl.BlockSpec((ATTR, HID), lambda i: (0, 0)),
    ],
    out_specs=pl.BlockSpec((NC, BN, HH), lambda i: (0, i, 0)),
    out_shape=jax.ShapeDtypeStruct((NC, NP, HH), jnp.float32),
)


def kernel(x, pos, edge_index, period_vec, batch, elem_embed, W_embed, rbf_mu,
           W1, W2, W_sh, W_self, W_out, W_attr):
    f32 = jnp.float32
    src = edge_index[0].astype(jnp.int32)
    dst = edge_index[1].astype(jnp.int32)
    src3 = src.reshape(NW, NCH, CH)
    dst3 = dst.reshape(NW, NCH, CH)
    srcS = src.reshape(NS, NCHS, CH)
    dstS = dst.reshape(NS, NCHS, CH)
    xp = jnp.pad(x[:, 0].astype(jnp.int32), (0, NP - N))
    x3 = xp.reshape(NW, XCH, XCS)
    pos_pad = jnp.pad(pos.astype(f32), ((0, 0), (0, 13)))
    per_pad = jnp.pad(period_vec.astype(f32), ((0, 0), (0, 13)))

    dvec, x_attr = _make_sc_gather()(pos_pad, src3, dst3, x3,
                                     elem_embed.astype(f32))

    mu = rbf_mu.astype(f32).reshape(1, NB)
    perm = jnp.array([32 * (p // 32) + 16 * (p % 2) + (p % 32) // 2
                      for p in range(HID)], jnp.int32)
    w2p = W2.astype(f32)[:, :, perm]
    wsh0 = W_sh[:, 0, :].astype(f32)[:, perm].reshape(NCONV, 1, HID)
    wshp = jnp.zeros((NCONV, 16, HID), f32).at[:, 0:3, :].set(
        W_sh[:, 1:4, :].astype(f32))[:, :, perm]

    def gates(i):
        return _gates1(dvec, per_pad, mu, W1[i].astype(f32),
                       w2p[i], wsh0[i], wshp[i])

    h = _h0(x_attr, W_embed.astype(f32))
    sc_agg = _make_sc_agg()
    g = gates(0)
    for i in range(NCONV):
        agg = sc_agg(h, g, srcS, dstS)
        if i + 1 < NCONV:
            g = gates(i + 1)
        h = _update(h, agg, x_attr, W_self[i].astype(f32),
                    W_out[i].astype(f32), W_attr[i].astype(f32))
    return jnp.concatenate([h[0], h[1]], axis=1)[:N]


# speedup vs baseline: 1.0427x; 1.0427x over previous
import functools

import jax
import jax.numpy as jnp
from jax import lax
from jax.experimental import pallas as pl
from jax.experimental.pallas import tpu as pltpu
from jax.experimental.pallas import tpu_sc as plsc

N = 10000
E = 320000
HID = 128
HH = HID // 2
ATTR = 16
NB = 8
NCONV = 3
GAMMA = 10.0

NP = 10240
NC, NS = 2, 16
NW = NC * NS
CH = 80
K = 2
KA = 5
ECT = E // NW
NCH = ECT // CH
ECS = E // NS
NCHS = ECS // CH
XCT = NP // NW
XCH = 4
XCS = XCT // XCH
RPT = NP // NS

C0 = 0.28209479177387814
C1 = 0.4886025119029199


def _silu(v):
    return v / (1.0 + jnp.exp(-v))


def _sc_gather_body(pos_hbm, src3_hbm, dst3_hbm, x3_hbm, emb_hbm,
                    dvec_hbm, xattr_hbm,
                    src_t, dst_t, x_t, psrc, pdst, obuf, xbuf,
                    lsem, wsem):
    c = lax.axis_index("c")
    s = lax.axis_index("s")
    wid = c * NS + s
    ebase = wid * ECT

    pltpu.sync_copy(src3_hbm.at[wid], src_t)
    pltpu.sync_copy(dst3_hbm.at[wid], dst_t)
    pltpu.sync_copy(x3_hbm.at[wid], x_t)

    def group(gi, _):
        i0 = gi * KA
        descs = []
        for b in range(KA):
            descs.append(pltpu.async_copy(
                pos_hbm.at[src_t.at[i0 + b]], psrc.at[b], lsem))
            descs.append(pltpu.async_copy(
                pos_hbm.at[dst_t.at[i0 + b]], pdst.at[b], lsem))
        for d in descs:
            d.wait()
        wdescs = []
        for b in range(KA):
            def row(r4, _):
                for rr in range(4):
                    r = r4 * 4 + rr
                    obuf[b, r] = pdst[b, r] - psrc[b, r]
                return 0

            lax.fori_loop(0, CH // 4, row, 0)
            wdescs.append(pltpu.async_copy(
                obuf.at[b], dvec_hbm.at[pl.ds(ebase + (i0 + b) * CH, CH)],
                wsem))
        for d in wdescs:
            d.wait()
        return 0

    lax.fori_loop(0, NCH // KA, group, 0)

    xbase = wid * XCT

    def xchunk(k, _):
        pltpu.async_copy(emb_hbm.at[x_t.at[k]], xbuf, lsem).wait()
        pltpu.sync_copy(xbuf, xattr_hbm.at[pl.ds(xbase + k * XCS, XCS)])
        return 0

    lax.fori_loop(0, XCH, xchunk, 0)


@functools.cache
def _make_sc_gather():
  return pl.kernel(
    _sc_gather_body,
    out_type=(jax.ShapeDtypeStruct((E, 16), jnp.float32),
              jax.ShapeDtypeStruct((NP, ATTR), jnp.float32)),
    mesh=plsc.VectorSubcoreMesh(core_axis_name="c", subcore_axis_name="s"),
    compiler_params=pltpu.CompilerParams(use_tc_tiling_on_sc=False),
    scratch_types=(
        pltpu.VMEM((NCH, CH), jnp.int32),
        pltpu.VMEM((NCH, CH), jnp.int32),
        pltpu.VMEM((XCH, XCS), jnp.int32),
        pltpu.VMEM((KA, CH, 16), jnp.float32),
        pltpu.VMEM((KA, CH, 16), jnp.float32),
        pltpu.VMEM((KA, CH, 16), jnp.float32),
        pltpu.VMEM((XCS, ATTR), jnp.float32),
        pltpu.SemaphoreType.DMA,
        pltpu.SemaphoreType.DMA,
    ),
  )


def _sc_agg_body(h_hbm, g_hbm, src3_hbm, dst3_hbm,
                 out_hbm,
                 src_t, dst_t, hb0, hb1, gb0, gb1, zbuf, agg_s,
                 lsem, ssem):
    hbl = (hb0, hb1)
    gbl = (gb0, gb1)
    c = lax.axis_index("c")
    s = lax.axis_index("s")
    ebase = s * ECS

    zv = jnp.zeros((16,), jnp.float32)

    def zrow(r, _):
        for j in range(HH // 16):
            zbuf[r, pl.ds(j * 16, 16)] = zv
        return 0

    lax.fori_loop(0, CH, zrow, 0)
    for t in range(RPT // CH):
        pltpu.sync_copy(zbuf, agg_s.at[pl.ds(s * RPT + t * CH, CH)])
    plsc.subcore_barrier()

    pltpu.sync_copy(src3_hbm.at[s], src_t)
    pltpu.sync_copy(dst3_hbm.at[s], dst_t)

    def group(gi, _):
        i0 = gi * K
        descs = []
        for b in range(K):
            descs.append(pltpu.async_copy(
                h_hbm.at[c].at[src_t.at[i0 + b]], hbl[b], lsem))
            descs.append(pltpu.async_copy(
                g_hbm.at[c, pl.ds(ebase + (i0 + b) * CH, CH)],
                gbl[b], lsem))
        for d in descs:
            d.wait()
        for b in range(K):
            def row(r2, _):
                for rr in range(2):
                    r = r2 * 2 + rr
                    for j in range(HH // 32):
                        v = gbl[b][r, pl.ds(j * 16, 16)]
                        a0 = plsc.bitcast(v << 16, jnp.float32)
                        a1 = plsc.bitcast(v & jnp.int32(-65536), jnp.float32)
                        sl0 = pl.ds(j * 32, 16)
                        sl1 = pl.ds(j * 32 + 16, 16)
                        hbl[b][r, sl0] = hbl[b][r, sl0] * a0
                        hbl[b][r, sl1] = hbl[b][r, sl1] * a1
                return 0

            lax.fori_loop(0, CH // 2, row, 0)
            pltpu.sync_copy(hbl[b], agg_s.at[dst_t.at[i0 + b]], add=True)
        return 0

    lax.fori_loop(0, NCHS // K, group, 0)
    plsc.subcore_barrier()
    pltpu.sync_copy(agg_s.at[pl.ds(s * RPT, RPT)],
                    out_hbm.at[c, pl.ds(s * RPT, RPT)])


@functools.cache
def _make_sc_agg():
  return pl.kernel(
    _sc_agg_body,
    out_type=jax.ShapeDtypeStruct((NC, NP, HH), jnp.float32),
    mesh=plsc.VectorSubcoreMesh(core_axis_name="c", subcore_axis_name="s"),
    compiler_params=pltpu.CompilerParams(use_tc_tiling_on_sc=False,
                                         needs_layout_passes=False),
    scratch_types=(
        pltpu.VMEM((NCHS, CH), jnp.int32),
        pltpu.VMEM((NCHS, CH), jnp.int32),
        pltpu.VMEM((CH, HH), jnp.float32),
        pltpu.VMEM((CH, HH), jnp.float32),
        pltpu.VMEM((CH, HH // 2), jnp.int32),
        pltpu.VMEM((CH, HH // 2), jnp.int32),
        pltpu.VMEM((CH, HH), jnp.float32),
        pltpu.VMEM_SHARED((NP, HH), jnp.float32),
        pltpu.SemaphoreType.DMA,
        pltpu.SemaphoreType.DMA,
    ),
  )


BE = 2000


def _gate_body(dvec_ref, per_ref, mu_ref, W1_ref, W2_ref, Wsh0_ref, Wshp_ref,
               g_ref):
    dv = dvec_ref[...] + per_ref[...]
    r2 = jnp.sum(dv * dv, axis=1, keepdims=True) + 1e-12
    r = jnp.sqrt(r2)
    up = dv / r
    rbf = jnp.exp(-GAMMA * (r - mu_ref[...]) ** 2)
    q = _silu(jnp.dot(rbf, W1_ref[...], preferred_element_type=jnp.float32))
    rad = jnp.dot(q, W2_ref[...], preferred_element_type=jnp.float32)
    shw = C0 * Wsh0_ref[...] + C1 * jnp.dot(
        up, Wshp_ref[...], preferred_element_type=jnp.float32)
    g = rad * shw
    gi = jax.lax.bitcast_convert_type(g, jnp.int32) + jnp.int32(32768)
    la = jax.lax.shift_right_logical(gi, 16)
    hb = gi & jnp.int32(-65536)
    w0 = hb[:, 32:64] | la[:, 0:32]
    w1 = hb[:, 96:128] | la[:, 64:96]
    g_ref[...] = jnp.stack([w0, w1])


_gates1 = pl.pallas_call(
    _gate_body,
    grid=(E // BE,),
    in_specs=[
        pl.BlockSpec((BE, 16), lambda i: (i, 0)),
        pl.BlockSpec((BE, 16), lambda i: (i, 0)),
        pl.BlockSpec((1, NB), lambda i: (0, 0)),
        pl.BlockSpec((NB, HID), lambda i: (0, 0)),
        pl.BlockSpec((HID, HID), lambda i: (0, 0)),
        pl.BlockSpec((1, HID), lambda i: (0, 0)),
        pl.BlockSpec((16, HID), lambda i: (0, 0)),
    ],
    out_specs=pl.BlockSpec((NC, BE, HH // 2), lambda i: (0, i, 0)),
    out_shape=jax.ShapeDtypeStruct((NC, E, HH // 2), jnp.int32),
)


def _h0_body(xattr_ref, w_ref, h_ref):
    h = jnp.dot(xattr_ref[...], w_ref[...], preferred_element_type=jnp.float32)
    h_ref[...] = jnp.stack([h[:, :HH], h[:, HH:]])


_h0 = pl.pallas_call(
    _h0_body,
    out_shape=jax.ShapeDtypeStruct((NC, NP, HH), jnp.float32),
)


BN = 1280


def _update_body(h_ref, agg_ref, xattr_ref, ws_ref, wo_ref, wa_ref, out_ref):
    h = jnp.concatenate([h_ref[0], h_ref[1]], axis=1)
    agg = jnp.concatenate([agg_ref[0], agg_ref[1]], axis=1)
    v = (jnp.dot(h, ws_ref[...], preferred_element_type=jnp.float32)
         + jnp.dot(agg, wo_ref[...], preferred_element_type=jnp.float32)
         + jnp.dot(xattr_ref[...], wa_ref[...],
                   preferred_element_type=jnp.float32))
    hn = _silu(v)
    out_ref[...] = jnp.stack([hn[:, :HH], hn[:, HH:]])


_update = pl.pallas_call(
    _update_body,
    grid=(NP // BN,),
    in_specs=[
        pl.BlockSpec((NC, BN, HH), lambda i: (0, i, 0)),
        pl.BlockSpec((NC, BN, HH), lambda i: (0, i, 0)),
        pl.BlockSpec((BN, ATTR), lambda i: (i, 0)),
        pl.BlockSpec((HID, HID), lambda i: (0, 0)),
        pl.BlockSpec((HID, HID), lambda i: (0, 0)),
        pl.BlockSpec((ATTR, HID), lambda i: (0, 0)),
    ],
    out_specs=pl.BlockSpec((NC, BN, HH), lambda i: (0, i, 0)),
    out_shape=jax.ShapeDtypeStruct((NC, NP, HH), jnp.float32),
)


def kernel(x, pos, edge_index, period_vec, batch, elem_embed, W_embed, rbf_mu,
           W1, W2, W_sh, W_self, W_out, W_attr):
    f32 = jnp.float32
    src = edge_index[0].astype(jnp.int32)
    dst = edge_index[1].astype(jnp.int32)
    src3 = src.reshape(NW, NCH, CH)
    dst3 = dst.reshape(NW, NCH, CH)
    srcS = src.reshape(NS, NCHS, CH)
    dstS = dst.reshape(NS, NCHS, CH)
    xp = jnp.pad(x[:, 0].astype(jnp.int32), (0, NP - N))
    x3 = xp.reshape(NW, XCH, XCS)
    pos_pad = jnp.pad(pos.astype(f32), ((0, 0), (0, 13)))
    per_pad = jnp.pad(period_vec.astype(f32), ((0, 0), (0, 13)))

    dvec, x_attr = _make_sc_gather()(pos_pad, src3, dst3, x3,
                                     elem_embed.astype(f32))

    mu = rbf_mu.astype(f32).reshape(1, NB)
    perm = jnp.array(
        [64 * (p // 64) + 32 * ((p % 32) // 16) + 16 * ((p % 64) // 32)
         + p % 16 for p in range(HID)], jnp.int32)
    w2p = W2.astype(f32)[:, :, perm]
    wsh0 = W_sh[:, 0, :].astype(f32)[:, perm].reshape(NCONV, 1, HID)
    wshp = jnp.zeros((NCONV, 16, HID), f32).at[:, 0:3, :].set(
        W_sh[:, 1:4, :].astype(f32))[:, :, perm]

    def gates(i):
        return _gates1(dvec, per_pad, mu, W1[i].astype(f32),
                       w2p[i], wsh0[i], wshp[i])

    h = _h0(x_attr, W_embed.astype(f32))
    sc_agg = _make_sc_agg()
    g = gates(0)
    for i in range(NCONV):
        agg = sc_agg(h, g, srcS, dstS)
        if i + 1 < NCONV:
            g = gates(i + 1)
        h = _update(h, agg, x_attr, W_self[i].astype(f32),
                    W_out[i].astype(f32), W_attr[i].astype(f32))
    return jnp.concatenate([h[0], h[1]], axis=1)[:N]


# speedup vs baseline: 1.1318x; 1.0855x over previous
import functools

import jax
import jax.numpy as jnp
from jax import lax
from jax.experimental import pallas as pl
from jax.experimental.pallas import tpu as pltpu
from jax.experimental.pallas import tpu_sc as plsc

N = 10000
E = 320000
HID = 128
HH = HID // 2
ATTR = 16
NB = 8
NCONV = 3
GAMMA = 10.0

NP = 10240
NC, NS = 2, 16
NW = NC * NS
CH = 100
K = 2
KA = 5
CHA = 80
ECT = E // NW
NCH = ECT // CHA
NCHA = NCH
ECS = E // NS
NCHS = ECS // CH
XCT = NP // NW
XCH = 4
XCHA = XCH
XCS = XCT // XCH
RPT = NP // NS

C0 = 0.28209479177387814
C1 = 0.4886025119029199


def _silu(v):
    return v / (1.0 + jnp.exp(-v))


def _sc_gather_body(pos_hbm, src3_hbm, dst3_hbm, x3_hbm, emb_hbm,
                    dvec_hbm, xattr_hbm,
                    src_t, dst_t, x_t, psrc, pdst, obuf, xbuf,
                    lsem, wsem):
    c = lax.axis_index("c")
    s = lax.axis_index("s")
    wid = c * NS + s
    ebase = wid * ECT

    pltpu.sync_copy(src3_hbm.at[wid], src_t)
    pltpu.sync_copy(dst3_hbm.at[wid], dst_t)
    pltpu.sync_copy(x3_hbm.at[wid], x_t)

    def group(gi, _):
        i0 = gi * KA
        descs = []
        for b in range(KA):
            descs.append(pltpu.async_copy(
                pos_hbm.at[src_t.at[i0 + b]], psrc.at[b], lsem))
            descs.append(pltpu.async_copy(
                pos_hbm.at[dst_t.at[i0 + b]], pdst.at[b], lsem))
        for d in descs:
            d.wait()
        wdescs = []
        for b in range(KA):
            def row(r4, _):
                for rr in range(4):
                    r = r4 * 4 + rr
                    obuf[b, r] = pdst[b, r] - psrc[b, r]
                return 0

            lax.fori_loop(0, CHA // 4, row, 0)
            wdescs.append(pltpu.async_copy(
                obuf.at[b], dvec_hbm.at[pl.ds(ebase + (i0 + b) * CHA, CHA)],
                wsem))
        for d in wdescs:
            d.wait()
        return 0

    lax.fori_loop(0, NCHA // KA, group, 0)

    xbase = wid * XCT

    def xchunk(k, _):
        pltpu.async_copy(emb_hbm.at[x_t.at[k]], xbuf, lsem).wait()
        pltpu.sync_copy(xbuf, xattr_hbm.at[pl.ds(xbase + k * XCS, XCS)])
        return 0

    lax.fori_loop(0, XCHA, xchunk, 0)


@functools.cache
def _make_sc_gather():
  return pl.kernel(
    _sc_gather_body,
    out_type=(jax.ShapeDtypeStruct((E, 16), jnp.float32),
              jax.ShapeDtypeStruct((NP, ATTR), jnp.float32)),
    mesh=plsc.VectorSubcoreMesh(core_axis_name="c", subcore_axis_name="s"),
    compiler_params=pltpu.CompilerParams(use_tc_tiling_on_sc=False),
    scratch_types=(
        pltpu.VMEM((NCHA, CHA), jnp.int32),
        pltpu.VMEM((NCHA, CHA), jnp.int32),
        pltpu.VMEM((XCHA, XCS), jnp.int32),
        pltpu.VMEM((KA, CHA, 16), jnp.float32),
        pltpu.VMEM((KA, CHA, 16), jnp.float32),
        pltpu.VMEM((KA, CHA, 16), jnp.float32),
        pltpu.VMEM((XCS, ATTR), jnp.float32),
        pltpu.SemaphoreType.DMA,
        pltpu.SemaphoreType.DMA,
    ),
  )


def _sc_agg_body(h_hbm, g_hbm, src3_hbm, dst3_hbm,
                 out_hbm,
                 src_t, dst_t, hb0, hb1, gb0, gb1, zbuf, agg_s,
                 lsem, ssem):
    hbl = (hb0, hb1)
    gbl = (gb0, gb1)
    c = lax.axis_index("c")
    s = lax.axis_index("s")
    ebase = s * ECS

    zv = jnp.zeros((16,), jnp.float32)

    def zrow(r, _):
        for j in range(HH // 16):
            zbuf[r, pl.ds(j * 16, 16)] = zv
        return 0

    lax.fori_loop(0, CH, zrow, 0)
    for t in range(6):
        pltpu.sync_copy(zbuf, agg_s.at[pl.ds(s * RPT + t * CH, CH)])
    pltpu.sync_copy(zbuf.at[pl.ds(0, RPT - 6 * CH)],
                    agg_s.at[pl.ds(s * RPT + 6 * CH, RPT - 6 * CH)])
    plsc.subcore_barrier()

    pltpu.sync_copy(src3_hbm.at[s], src_t)
    pltpu.sync_copy(dst3_hbm.at[s], dst_t)

    def group(gi, _):
        i0 = gi * K
        descs = []
        for b in range(K):
            descs.append(pltpu.async_copy(
                h_hbm.at[c].at[src_t.at[i0 + b]], hbl[b], lsem))
            descs.append(pltpu.async_copy(
                g_hbm.at[c, pl.ds(ebase + (i0 + b) * CH, CH)],
                gbl[b], lsem))
        for d in descs:
            d.wait()
        sdescs = []
        for b in range(K):
            def row(r2, _):
                for rr in range(2):
                    r = r2 * 2 + rr
                    for j in range(HH // 16):
                        sl = pl.ds(j * 16, 16)
                        hbl[b][r, sl] = hbl[b][r, sl] * gbl[b][r, sl]
                return 0

            lax.fori_loop(0, CH // 2, row, 0)
            sdescs.append(pltpu.async_copy(
                hbl[b], agg_s.at[dst_t.at[i0 + b]], ssem, add=True))
        for d in sdescs:
            d.wait()
        return 0

    lax.fori_loop(0, NCHS // K, group, 0)
    plsc.subcore_barrier()
    pltpu.sync_copy(agg_s.at[pl.ds(s * RPT, RPT)],
                    out_hbm.at[c, pl.ds(s * RPT, RPT)])


@functools.cache
def _make_sc_agg():
  return pl.kernel(
    _sc_agg_body,
    out_type=jax.ShapeDtypeStruct((NC, NP, HH), jnp.float32),
    mesh=plsc.VectorSubcoreMesh(core_axis_name="c", subcore_axis_name="s"),
    compiler_params=pltpu.CompilerParams(use_tc_tiling_on_sc=False),
    scratch_types=(
        pltpu.VMEM((NCHS, CH), jnp.int32),
        pltpu.VMEM((NCHS, CH), jnp.int32),
        pltpu.VMEM((CH, HH), jnp.float32),
        pltpu.VMEM((CH, HH), jnp.float32),
        pltpu.VMEM((CH, HH), jnp.float32),
        pltpu.VMEM((CH, HH), jnp.float32),
        pltpu.VMEM((CH, HH), jnp.float32),
        pltpu.VMEM_SHARED((NP, HH), jnp.float32),
        pltpu.SemaphoreType.DMA,
        pltpu.SemaphoreType.DMA,
    ),
  )


BE = 2000


def _gate_body(dvec_ref, per_ref, mu_ref, W1_ref, W2_ref, Wsh0_ref, Wshp_ref,
               g_ref):
    dv = dvec_ref[...] + per_ref[...]
    r2 = jnp.sum(dv * dv, axis=1, keepdims=True) + 1e-12
    r = jnp.sqrt(r2)
    up = dv / r
    rbf = jnp.exp(-GAMMA * (r - mu_ref[...]) ** 2)
    q = _silu(jnp.dot(rbf, W1_ref[...], preferred_element_type=jnp.float32))
    rad = jnp.dot(q, W2_ref[...], preferred_element_type=jnp.float32)
    shw = C0 * Wsh0_ref[...] + C1 * jnp.dot(
        up, Wshp_ref[...], preferred_element_type=jnp.float32)
    g = rad * shw
    g_ref[...] = jnp.stack([g[:, :HH], g[:, HH:]])


_gates1 = pl.pallas_call(
    _gate_body,
    grid=(E // BE,),
    in_specs=[
        pl.BlockSpec((BE, 16), lambda i: (i, 0)),
        pl.BlockSpec((BE, 16), lambda i: (i, 0)),
        pl.BlockSpec((1, NB), lambda i: (0, 0)),
        pl.BlockSpec((NB, HID), lambda i: (0, 0)),
        pl.BlockSpec((HID, HID), lambda i: (0, 0)),
        pl.BlockSpec((1, HID), lambda i: (0, 0)),
        pl.BlockSpec((16, HID), lambda i: (0, 0)),
    ],
    out_specs=pl.BlockSpec((NC, BE, HH), lambda i: (0, i, 0)),
    out_shape=jax.ShapeDtypeStruct((NC, E, HH), jnp.float32),
)


def _h0_body(xattr_ref, w_ref, h_ref):
    h = jnp.dot(xattr_ref[...], w_ref[...], preferred_element_type=jnp.float32)
    h_ref[...] = jnp.stack([h[:, :HH], h[:, HH:]])


_h0 = pl.pallas_call(
    _h0_body,
    out_shape=jax.ShapeDtypeStruct((NC, NP, HH), jnp.float32),
)


BN = 1280


def _update_body(h_ref, agg_ref, xattr_ref, ws_ref, wo_ref, wa_ref, out_ref):
    h = jnp.concatenate([h_ref[0], h_ref[1]], axis=1)
    agg = jnp.concatenate([agg_ref[0], agg_ref[1]], axis=1)
    v = (jnp.dot(h, ws_ref[...], preferred_element_type=jnp.float32)
         + jnp.dot(agg, wo_ref[...], preferred_element_type=jnp.float32)
         + jnp.dot(xattr_ref[...], wa_ref[...],
                   preferred_element_type=jnp.float32))
    hn = _silu(v)
    out_ref[...] = jnp.stack([hn[:, :HH], hn[:, HH:]])


_update = pl.pallas_call(
    _update_body,
    grid=(NP // BN,),
    in_specs=[
        pl.BlockSpec((NC, BN, HH), lambda i: (0, i, 0)),
        pl.BlockSpec((NC, BN, HH), lambda i: (0, i, 0)),
        pl.BlockSpec((BN, ATTR), lambda i: (i, 0)),
        pl.BlockSpec((HID, HID), lambda i: (0, 0)),
        pl.BlockSpec((HID, HID), lambda i: (0, 0)),
        pl.BlockSpec((ATTR, HID), lambda i: (0, 0)),
    ],
    out_specs=pl.BlockSpec((NC, BN, HH), lambda i: (0, i, 0)),
    out_shape=jax.ShapeDtypeStruct((NC, NP, HH), jnp.float32),
)


def kernel(x, pos, edge_index, period_vec, batch, elem_embed, W_embed, rbf_mu,
           W1, W2, W_sh, W_self, W_out, W_attr):
    f32 = jnp.float32
    src = edge_index[0].astype(jnp.int32)
    dst = edge_index[1].astype(jnp.int32)
    src3 = src.reshape(NW, NCH, CHA)
    dst3 = dst.reshape(NW, NCH, CHA)
    srcS = src.reshape(NS, NCHS, CH)
    dstS = dst.reshape(NS, NCHS, CH)
    xp = jnp.pad(x[:, 0].astype(jnp.int32), (0, NP - N))
    x3 = xp.reshape(NW, XCH, XCS)
    pos_pad = jnp.pad(pos.astype(f32), ((0, 0), (0, 13)))
    per_pad = jnp.pad(period_vec.astype(f32), ((0, 0), (0, 13)))

    dvec, x_attr = _make_sc_gather()(pos_pad, src3, dst3, x3,
                                     elem_embed.astype(f32))

    mu = rbf_mu.astype(f32).reshape(1, NB)
    wsh0 = W_sh[:, 0, :].astype(f32).reshape(NCONV, 1, HID)
    wshp = jnp.zeros((NCONV, 16, HID), f32).at[:, 0:3, :].set(
        W_sh[:, 1:4, :].astype(f32))

    def gates(i):
        return _gates1(dvec, per_pad, mu, W1[i].astype(f32),
                       W2[i].astype(f32), wsh0[i], wshp[i])

    h = _h0(x_attr, W_embed.astype(f32))
    sc_agg = _make_sc_agg()
    g = gates(0)
    for i in range(NCONV):
        agg = sc_agg(h, g, srcS, dstS)
        if i + 1 < NCONV:
            g = gates(i + 1)
        h = _update(h, agg, x_attr, W_self[i].astype(f32),
                    W_out[i].astype(f32), W_attr[i].astype(f32))
    return jnp.concatenate([h[0], h[1]], axis=1)[:N]


# speedup vs baseline: 1.1376x; 1.0051x over previous
import functools

import jax
import jax.numpy as jnp
from jax import lax
from jax.experimental import pallas as pl
from jax.experimental.pallas import tpu as pltpu
from jax.experimental.pallas import tpu_sc as plsc

N = 10000
E = 320000
HID = 128
HH = HID // 2
ATTR = 16
NB = 8
NCONV = 3
GAMMA = 10.0

NP = 10240
NC, NS = 2, 16
NW = NC * NS
CH = 125
K = 2
KA = 5
CHA = 80
ECT = E // NW
NCH = ECT // CHA
NCHA = NCH
ECS = E // NS
NCHS = ECS // CH
XCT = NP // NW
XCH = 4
XCHA = XCH
XCS = XCT // XCH
RPT = NP // NS

C0 = 0.28209479177387814
C1 = 0.4886025119029199


def _silu(v):
    return v / (1.0 + jnp.exp(-v))


def _sc_gather_body(pos_hbm, src3_hbm, dst3_hbm, x3_hbm, emb_hbm,
                    dvec_hbm, xattr_hbm,
                    src_t, dst_t, x_t, psrc, pdst, obuf, xbuf,
                    lsem, wsem):
    c = lax.axis_index("c")
    s = lax.axis_index("s")
    wid = c * NS + s
    ebase = wid * ECT

    pltpu.sync_copy(src3_hbm.at[wid], src_t)
    pltpu.sync_copy(dst3_hbm.at[wid], dst_t)
    pltpu.sync_copy(x3_hbm.at[wid], x_t)

    def group(gi, _):
        i0 = gi * KA
        descs = []
        for b in range(KA):
            descs.append(pltpu.async_copy(
                pos_hbm.at[src_t.at[i0 + b]], psrc.at[b], lsem))
            descs.append(pltpu.async_copy(
                pos_hbm.at[dst_t.at[i0 + b]], pdst.at[b], lsem))
        for d in descs:
            d.wait()
        wdescs = []
        for b in range(KA):
            def row(r4, _):
                for rr in range(4):
                    r = r4 * 4 + rr
                    obuf[b, r] = pdst[b, r] - psrc[b, r]
                return 0

            lax.fori_loop(0, CHA // 4, row, 0)
            wdescs.append(pltpu.async_copy(
                obuf.at[b], dvec_hbm.at[pl.ds(ebase + (i0 + b) * CHA, CHA)],
                wsem))
        for d in wdescs:
            d.wait()
        return 0

    lax.fori_loop(0, NCHA // KA, group, 0)

    xbase = wid * XCT

    def xchunk(k, _):
        pltpu.async_copy(emb_hbm.at[x_t.at[k]], xbuf, lsem).wait()
        pltpu.sync_copy(xbuf, xattr_hbm.at[pl.ds(xbase + k * XCS, XCS)])
        return 0

    lax.fori_loop(0, XCHA, xchunk, 0)


@functools.cache
def _make_sc_gather():
  return pl.kernel(
    _sc_gather_body,
    out_type=(jax.ShapeDtypeStruct((E, 16), jnp.float32),
              jax.ShapeDtypeStruct((NP, ATTR), jnp.float32)),
    mesh=plsc.VectorSubcoreMesh(core_axis_name="c", subcore_axis_name="s"),
    compiler_params=pltpu.CompilerParams(use_tc_tiling_on_sc=False),
    scratch_types=(
        pltpu.VMEM((NCHA, CHA), jnp.int32),
        pltpu.VMEM((NCHA, CHA), jnp.int32),
        pltpu.VMEM((XCHA, XCS), jnp.int32),
        pltpu.VMEM((KA, CHA, 16), jnp.float32),
        pltpu.VMEM((KA, CHA, 16), jnp.float32),
        pltpu.VMEM((KA, CHA, 16), jnp.float32),
        pltpu.VMEM((XCS, ATTR), jnp.float32),
        pltpu.SemaphoreType.DMA,
        pltpu.SemaphoreType.DMA,
    ),
  )


def _sc_agg_body(h_hbm, g_hbm, src3_hbm, dst3_hbm,
                 out_hbm,
                 src_t, dst_t, hb0, hb1, gb0, gb1, zbuf, agg_s,
                 lsem, ssem):
    hbl = (hb0, hb1)
    gbl = (gb0, gb1)
    c = lax.axis_index("c")
    s = lax.axis_index("s")
    ebase = s * ECS

    zv = jnp.zeros((16,), jnp.float32)

    def zrow(r, _):
        for j in range(HH // 16):
            zbuf[r, pl.ds(j * 16, 16)] = zv
        return 0

    lax.fori_loop(0, CH, zrow, 0)
    for t in range(5):
        pltpu.sync_copy(zbuf, agg_s.at[pl.ds(s * RPT + t * CH, CH)])
    pltpu.sync_copy(zbuf.at[pl.ds(0, RPT - 5 * CH)],
                    agg_s.at[pl.ds(s * RPT + 5 * CH, RPT - 5 * CH)])
    plsc.subcore_barrier()

    pltpu.sync_copy(src3_hbm.at[s], src_t)
    pltpu.sync_copy(dst3_hbm.at[s], dst_t)

    def group(gi, _):
        i0 = gi * K
        descs = []
        for b in range(K):
            descs.append(pltpu.async_copy(
                h_hbm.at[c].at[src_t.at[i0 + b]], hbl[b], lsem))
            descs.append(pltpu.async_copy(
                g_hbm.at[c, pl.ds(ebase + (i0 + b) * CH, CH)],
                gbl[b], lsem))
        for d in descs:
            d.wait()
        sdescs = []
        for b in range(K):
            def row(r5, _):
                for rr in range(5):
                    r = r5 * 5 + rr
                    for j in range(HH // 16):
                        sl = pl.ds(j * 16, 16)
                        hbl[b][r, sl] = hbl[b][r, sl] * gbl[b][r, sl]
                return 0

            lax.fori_loop(0, CH // 5, row, 0)
            sdescs.append(pltpu.async_copy(
                hbl[b], agg_s.at[dst_t.at[i0 + b]], ssem, add=True))
        for d in sdescs:
            d.wait()
        return 0

    lax.fori_loop(0, NCHS // K, group, 0)
    plsc.subcore_barrier()
    pltpu.sync_copy(agg_s.at[pl.ds(s * RPT, RPT)],
                    out_hbm.at[c, pl.ds(s * RPT, RPT)])


@functools.cache
def _make_sc_agg():
  return pl.kernel(
    _sc_agg_body,
    out_type=jax.ShapeDtypeStruct((NC, NP, HH), jnp.float32),
    mesh=plsc.VectorSubcoreMesh(core_axis_name="c", subcore_axis_name="s"),
    compiler_params=pltpu.CompilerParams(use_tc_tiling_on_sc=False),
    scratch_types=(
        pltpu.VMEM((NCHS, CH), jnp.int32),
        pltpu.VMEM((NCHS, CH), jnp.int32),
        pltpu.VMEM((CH, HH), jnp.float32),
        pltpu.VMEM((CH, HH), jnp.float32),
        pltpu.VMEM((CH, HH), jnp.float32),
        pltpu.VMEM((CH, HH), jnp.float32),
        pltpu.VMEM((CH, HH), jnp.float32),
        pltpu.VMEM_SHARED((NP, HH), jnp.float32),
        pltpu.SemaphoreType.DMA,
        pltpu.SemaphoreType.DMA,
    ),
  )


BE = 2000


def _gate_body(dvec_ref, per_ref, mu_ref, W1_ref, W2_ref, Wsh0_ref, Wshp_ref,
               g_ref):
    dv = dvec_ref[...] + per_ref[...]
    r2 = jnp.sum(dv * dv, axis=1, keepdims=True) + 1e-12
    r = jnp.sqrt(r2)
    up = dv / r
    rbf = jnp.exp(-GAMMA * (r - mu_ref[...]) ** 2)
    q = _silu(jnp.dot(rbf, W1_ref[...], preferred_element_type=jnp.float32))
    rad = jnp.dot(q, W2_ref[...], preferred_element_type=jnp.float32)
    shw = C0 * Wsh0_ref[...] + C1 * jnp.dot(
        up, Wshp_ref[...], preferred_element_type=jnp.float32)
    g = rad * shw
    g_ref[...] = jnp.stack([g[:, :HH], g[:, HH:]])


_gates1 = pl.pallas_call(
    _gate_body,
    grid=(E // BE,),
    in_specs=[
        pl.BlockSpec((BE, 16), lambda i: (i, 0)),
        pl.BlockSpec((BE, 16), lambda i: (i, 0)),
        pl.BlockSpec((1, NB), lambda i: (0, 0)),
        pl.BlockSpec((NB, HID), lambda i: (0, 0)),
        pl.BlockSpec((HID, HID), lambda i: (0, 0)),
        pl.BlockSpec((1, HID), lambda i: (0, 0)),
        pl.BlockSpec((16, HID), lambda i: (0, 0)),
    ],
    out_specs=pl.BlockSpec((NC, BE, HH), lambda i: (0, i, 0)),
    out_shape=jax.ShapeDtypeStruct((NC, E, HH), jnp.float32),
)


def _h0_body(xattr_ref, w_ref, h_ref):
    h = jnp.dot(xattr_ref[...], w_ref[...], preferred_element_type=jnp.float32)
    h_ref[...] = jnp.stack([h[:, :HH], h[:, HH:]])


_h0 = pl.pallas_call(
    _h0_body,
    out_shape=jax.ShapeDtypeStruct((NC, NP, HH), jnp.float32),
)


BN = 1280


def _update_body(h_ref, agg_ref, xattr_ref, ws_ref, wo_ref, wa_ref, out_ref):
    h = jnp.concatenate([h_ref[0], h_ref[1]], axis=1)
    agg = jnp.concatenate([agg_ref[0], agg_ref[1]], axis=1)
    v = (jnp.dot(h, ws_ref[...], preferred_element_type=jnp.float32)
         + jnp.dot(agg, wo_ref[...], preferred_element_type=jnp.float32)
         + jnp.dot(xattr_ref[...], wa_ref[...],
                   preferred_element_type=jnp.float32))
    hn = _silu(v)
    out_ref[...] = jnp.stack([hn[:, :HH], hn[:, HH:]])


_update = pl.pallas_call(
    _update_body,
    grid=(NP // BN,),
    in_specs=[
        pl.BlockSpec((NC, BN, HH), lambda i: (0, i, 0)),
        pl.BlockSpec((NC, BN, HH), lambda i: (0, i, 0)),
        pl.BlockSpec((BN, ATTR), lambda i: (i, 0)),
        pl.BlockSpec((HID, HID), lambda i: (0, 0)),
        pl.BlockSpec((HID, HID), lambda i: (0, 0)),
        pl.BlockSpec((ATTR, HID), lambda i: (0, 0)),
    ],
    out_specs=pl.BlockSpec((NC, BN, HH), lambda i: (0, i, 0)),
    out_shape=jax.ShapeDtypeStruct((NC, NP, HH), jnp.float32),
)


def kernel(x, pos, edge_index, period_vec, batch, elem_embed, W_embed, rbf_mu,
           W1, W2, W_sh, W_self, W_out, W_attr):
    f32 = jnp.float32
    src = edge_index[0].astype(jnp.int32)
    dst = edge_index[1].astype(jnp.int32)
    src3 = src.reshape(NW, NCH, CHA)
    dst3 = dst.reshape(NW, NCH, CHA)
    srcS = src.reshape(NS, NCHS, CH)
    dstS = dst.reshape(NS, NCHS, CH)
    xp = jnp.pad(x[:, 0].astype(jnp.int32), (0, NP - N))
    x3 = xp.reshape(NW, XCH, XCS)
    pos_pad = jnp.pad(pos.astype(f32), ((0, 0), (0, 13)))
    per_pad = jnp.pad(period_vec.astype(f32), ((0, 0), (0, 13)))

    dvec, x_attr = _make_sc_gather()(pos_pad, src3, dst3, x3,
                                     elem_embed.astype(f32))

    mu = rbf_mu.astype(f32).reshape(1, NB)
    wsh0 = W_sh[:, 0, :].astype(f32).reshape(NCONV, 1, HID)
    wshp = jnp.zeros((NCONV, 16, HID), f32).at[:, 0:3, :].set(
        W_sh[:, 1:4, :].astype(f32))

    def gates(i):
        return _gates1(dvec, per_pad, mu, W1[i].astype(f32),
                       W2[i].astype(f32), wsh0[i], wshp[i])

    h = _h0(x_attr, W_embed.astype(f32))
    sc_agg = _make_sc_agg()
    g = gates(0)
    for i in range(NCONV):
        agg = sc_agg(h, g, srcS, dstS)
        if i + 1 < NCONV:
            g = gates(i + 1)
        h = _update(h, agg, x_attr, W_self[i].astype(f32),
                    W_out[i].astype(f32), W_attr[i].astype(f32))
    return jnp.concatenate([h[0], h[1]], axis=1)[:N]
